# Initial kernel scaffold; baseline (speedup 1.0000x reference)
import jax
import jax.numpy as jnp
from jax.experimental import pallas as pl


def kernel(x, idx_token, agg_weight, W_conv, ln_gamma, ln_beta, W_score, b_score):
    raise NotImplementedError("write your pallas kernel here")


# speedup vs baseline: 18.0307x; 18.0307x over previous
import jax
import jax.numpy as jnp
from jax import lax
from jax.experimental import pallas as pl
from jax.experimental.pallas import tpu as pltpu
from jax.experimental.pallas import tpu_sc as plsc

_B, _N, _C = 8, 2048, 128
_K = 5
_CL = 2
_RB = 256
_SQRT_C = float(_C) ** 0.5


def _row(v):
    return jnp.reshape(v, (1, -1))


def _d2_block(xr_b, xf_b, sqr_col, sqf_row):
    dot = lax.dot_general(xr_b, xf_b, (((1,), (1,)), ((), ())),
                          preferred_element_type=jnp.float32)
    return sqr_col + sqf_row - 2.0 * dot


def _to_dist(d2):
    return jnp.sqrt(jnp.maximum(d2, 0.0)) / jnp.float32(_SQRT_C)


def _pass1_body(xnbr_ref, xnbf_ref, sqr_ref, sqf_ref, noise_ref,
                den_ref, dmax_ref):
    sqr_col = jnp.reshape(sqr_ref[0], (_RB, 1))
    d2 = _d2_block(xnbr_ref[0], xnbf_ref[0], sqr_col, sqf_ref[0])

    colid = lax.broadcasted_iota(jnp.int32, (_RB, _N), 1)
    work = d2
    total = jnp.zeros((_RB, 1), jnp.float32)
    for _ in range(_K):
        m = jnp.min(work, axis=1, keepdims=True)
        md = _to_dist(m)
        total = total + md * md
        jmin = jnp.min(jnp.where(work == m, colid, _N), axis=1, keepdims=True)
        work = jnp.where(colid == jmin, jnp.float32(jnp.inf), work)

    den_col = jnp.exp(-(total / jnp.float32(_K)))
    den_ref[0] = _row(den_col) + noise_ref[0]

    bm = jnp.max(d2, keepdims=True).reshape(1, 1, 1)
    rb = pl.program_id(1)

    @pl.when(rb == 0)
    def _():
        dmax_ref[...] = bm

    @pl.when(rb != 0)
    def _():
        dmax_ref[...] = jnp.maximum(dmax_ref[...], bm)


def _pass1(xnb, sq, noise):
    nrb = _N // _RB
    return pl.pallas_call(
        _pass1_body,
        grid=(_B, nrb),
        in_specs=[
            pl.BlockSpec((1, _RB, _C), lambda i, j: (i, j, 0)),
            pl.BlockSpec((1, _N, _C), lambda i, j: (i, 0, 0)),
            pl.BlockSpec((1, 1, _RB), lambda i, j: (i, 0, j)),
            pl.BlockSpec((1, 1, _N), lambda i, j: (i, 0, 0)),
            pl.BlockSpec((1, 1, _RB), lambda i, j: (i, 0, j)),
        ],
        out_specs=[
            pl.BlockSpec((1, 1, _RB), lambda i, j: (i, 0, j)),
            pl.BlockSpec((1, 1, 1), lambda i, j: (i, 0, 0)),
        ],
        out_shape=[
            jax.ShapeDtypeStruct((_B, 1, _N), jnp.float32),
            jax.ShapeDtypeStruct((_B, 1, 1), jnp.float32),
        ],
        compiler_params=pltpu.CompilerParams(
            dimension_semantics=("parallel", "arbitrary")),
    )(xnb, xnb, sq, sq, noise)


def _pass2_body(xnbr_ref, xnbf_ref, sqr_ref, sqf_ref, denf_ref, denr_ref,
                dmax_ref, out_ref):
    sqr_col = jnp.reshape(sqr_ref[0], (_RB, 1))
    d2 = _d2_block(xnbr_ref[0], xnbf_ref[0], sqr_col, sqf_ref[0])

    den_row = denf_ref[0]
    den_col = jnp.reshape(denr_ref[0], (_RB, 1))
    dmax2 = dmax_ref[0]
    mask = den_row > den_col
    dmin = _to_dist(jnp.min(jnp.where(mask, d2, dmax2), axis=1, keepdims=True))
    out_ref[0] = _row(dmin * den_col)


def _pass2(xnb, sq, den, dmax):
    nrb = _N // _RB
    return pl.pallas_call(
        _pass2_body,
        grid=(_B, nrb),
        in_specs=[
            pl.BlockSpec((1, _RB, _C), lambda i, j: (i, j, 0)),
            pl.BlockSpec((1, _N, _C), lambda i, j: (i, 0, 0)),
            pl.BlockSpec((1, 1, _RB), lambda i, j: (i, 0, j)),
            pl.BlockSpec((1, 1, _N), lambda i, j: (i, 0, 0)),
            pl.BlockSpec((1, 1, _N), lambda i, j: (i, 0, 0)),
            pl.BlockSpec((1, 1, _RB), lambda i, j: (i, 0, j)),
            pl.BlockSpec((1, 1, 1), lambda i, j: (i, 0, 0)),
        ],
        out_specs=pl.BlockSpec((1, 1, _RB), lambda i, j: (i, 0, j)),
        out_shape=jax.ShapeDtypeStruct((_B, 1, _N), jnp.float32),
        compiler_params=pltpu.CompilerParams(
            dimension_semantics=("parallel", "arbitrary")),
    )(xnb, xnb, sq, sq, den, den, dmax)


def _finalize_body(xn_ref, xnb_ref, sqf_ref, sc_ref, ts_ref,
                   xm_ref, cl_ref, nw_ref):
    s = sc_ref[0]
    colid = lax.broadcasted_iota(jnp.int32, (1, _N), 1)
    m0 = jnp.max(s, keepdims=True)
    i0 = jnp.min(jnp.where(s == m0, colid, _N), keepdims=True)
    s2 = jnp.where(colid == i0, -jnp.float32(jnp.inf), s)
    m1 = jnp.max(s2, keepdims=True)
    i1 = jnp.min(jnp.where(s2 == m1, colid, _N), keepdims=True)

    rowid2 = lax.broadcasted_iota(jnp.int32, (_CL, _N), 0)
    cid2 = lax.broadcasted_iota(jnp.int32, (_CL, _N), 1)
    sel = jnp.where(rowid2 == 0, i0, i1)
    oh = (cid2 == sel).astype(jnp.bfloat16)
    xnb = xnb_ref[0]
    xc = lax.dot_general(oh, xnb, (((1,), (0,)), ((), ())),
                         preferred_element_type=jnp.float32)

    sqf = sqf_ref[0]
    sq0 = jnp.sum(jnp.where(colid == i0, sqf, 0.0), keepdims=True)
    sq1 = jnp.sum(jnp.where(colid == i1, sqf, 0.0), keepdims=True)
    sqc_col = jnp.where(lax.broadcasted_iota(jnp.int32, (_CL, 1), 0) == 0,
                        sq0, sq1)

    dot2 = lax.dot_general(xc.astype(jnp.bfloat16), xnb,
                           (((1,), (1,)), ((), ())),
                           preferred_element_type=jnp.float32)
    d2 = sqc_col + sqf - 2.0 * dot2
    dsel = jnp.sqrt(jnp.maximum(d2, 0.0)) / jnp.float32(_SQRT_C)

    cl = (dsel[1:2, :] < dsel[0:1, :]).astype(jnp.int32)
    cl = jnp.where(colid == i0, 0, cl)
    cl = jnp.where(colid == i1, 1, cl)

    tw = jnp.exp(ts_ref[0])
    c1 = cl.astype(jnp.float32)
    c0 = 1.0 - c1
    aw0 = jnp.sum(tw * c0, keepdims=True) + 1e-6
    aw1 = jnp.sum(tw * c1, keepdims=True) + 1e-6
    nw = tw / jnp.where(cl == 1, aw1, aw0)

    wmat = jnp.concatenate([nw * c0, nw * c1], axis=0)
    xm = lax.dot_general(wmat, xn_ref[0], (((1,), (0,)), ((), ())),
                         precision=lax.Precision.HIGHEST,
                         preferred_element_type=jnp.float32)

    xm_ref[0] = xm
    cl_ref[0] = cl
    nw_ref[0] = nw


def _finalize(xn, xnb, sq, score_c, ts):
    return pl.pallas_call(
        _finalize_body,
        grid=(_B,),
        in_specs=[
            pl.BlockSpec((1, _N, _C), lambda i: (i, 0, 0)),
            pl.BlockSpec((1, _N, _C), lambda i: (i, 0, 0)),
            pl.BlockSpec((1, 1, _N), lambda i: (i, 0, 0)),
            pl.BlockSpec((1, 1, _N), lambda i: (i, 0, 0)),
            pl.BlockSpec((1, 1, _N), lambda i: (i, 0, 0)),
        ],
        out_specs=[
            pl.BlockSpec((1, _CL, _C), lambda i: (i, 0, 0)),
            pl.BlockSpec((1, 1, _N), lambda i: (i, 0, 0)),
            pl.BlockSpec((1, 1, _N), lambda i: (i, 0, 0)),
        ],
        out_shape=[
            jax.ShapeDtypeStruct((_B, _CL, _C), jnp.float32),
            jax.ShapeDtypeStruct((_B, 1, _N), jnp.int32),
            jax.ShapeDtypeStruct((_B, 1, _N), jnp.float32),
        ],
    )(xn, xnb, sq, score_c, ts)


_NW = 32
_CHUNK = _B * _N // _NW
_NC = 2


def _sc_gather_body(idx_hbm, cl_hbm, nw_hbm, agg_hbm, itn_hbm, awn_hbm,
                    idx_v, cl_v, nw_v, agg_v, oti_v, oaw_v):
    wid = lax.axis_index("s") * _NC + lax.axis_index("c")
    base = wid * _CHUNK
    b = base // _N
    pltpu.sync_copy(cl_hbm.at[b], cl_v)
    pltpu.sync_copy(nw_hbm.at[b], nw_v)
    pltpu.sync_copy(idx_hbm.at[pl.ds(base, _CHUNK)], idx_v)
    pltpu.sync_copy(agg_hbm.at[pl.ds(base, _CHUNK)], agg_v)
    for j in range(_CHUNK // 16):
        sl = pl.ds(j * 16, 16)
        ii = idx_v[sl]
        oti_v[sl] = plsc.load_gather(cl_v, [ii])
        oaw_v[sl] = plsc.load_gather(nw_v, [ii]) * agg_v[sl]
    pltpu.sync_copy(oti_v, itn_hbm.at[pl.ds(base, _CHUNK)])
    pltpu.sync_copy(oaw_v, awn_hbm.at[pl.ds(base, _CHUNK)])


def _sc_gather(idx_flat, cl2d, nw2d, agg_flat):
    mesh = plsc.VectorSubcoreMesh(core_axis_name="c", subcore_axis_name="s")
    k = pl.kernel(
        _sc_gather_body,
        mesh=mesh,
        out_type=[
            jax.ShapeDtypeStruct((_B * _N,), jnp.int32),
            jax.ShapeDtypeStruct((_B * _N,), jnp.float32),
        ],
        scratch_types=[
            pltpu.VMEM((_CHUNK,), jnp.int32),
            pltpu.VMEM((_N,), jnp.int32),
            pltpu.VMEM((_N,), jnp.float32),
            pltpu.VMEM((_CHUNK,), jnp.float32),
            pltpu.VMEM((_CHUNK,), jnp.int32),
            pltpu.VMEM((_CHUNK,), jnp.float32),
        ],
        compiler_params=pltpu.CompilerParams(needs_layout_passes=False),
    )
    return k(idx_flat, cl2d, nw2d, agg_flat)


def kernel(x, idx_token, agg_weight, W_conv, ln_gamma, ln_beta,
           W_score, b_score):
    xt = jnp.transpose(x, (0, 2, 1))
    conv = lax.conv_general_dilated(xt, W_conv, (1,), ((1, 1),),
                                    dimension_numbers=("NCH", "OIH", "NCH"))
    xr = x + jnp.transpose(conv, (0, 2, 1))
    mu = jnp.mean(xr, -1, keepdims=True)
    var = jnp.var(xr, -1, keepdims=True)
    xn = (xr - mu) / jnp.sqrt(var + 1e-5) * ln_gamma + ln_beta
    token_score = xn @ W_score.T + b_score
    sq = jnp.sum(xn * xn, -1).reshape(_B, 1, _N)
    xnb = xn.astype(jnp.bfloat16)
    ts_row = token_score.reshape(_B, 1, _N)

    noise = (jax.random.uniform(jax.random.key(42), (_B, _N),
                                dtype=jnp.float32) * 1e-6).reshape(_B, 1, _N)
    den, dmax = _pass1(xnb, sq, noise)
    score_c = _pass2(xnb, sq, den, dmax)
    xm, cl, nw = _finalize(xn, xnb, sq, score_c, ts_row)

    itn, awn = _sc_gather(idx_token.reshape(-1),
                          cl.reshape(_B, _N),
                          nw.reshape(_B, _N),
                          agg_weight.reshape(-1))

    x_merged = xm
    idx_token_new = itn.reshape(_B, _N)
    agg_weight_new = awn.reshape(_B, _N, 1)
    return x_merged, idx_token_new, agg_weight_new, token_score
